# Initial kernel scaffold; baseline (speedup 1.0000x reference)
import jax
import jax.numpy as jnp
from jax.experimental import pallas as pl


def kernel(edge_nodes, edge_rels, node_embeddings, weights1, weights2, bias1, bias2):
    raise NotImplementedError("write your pallas kernel here")


# speedup vs baseline: 22.9605x; 22.9605x over previous
import functools

import jax
import jax.numpy as jnp
from jax import lax
from jax.experimental import pallas as pl
from jax.experimental.pallas import tpu as pltpu
from jax.experimental.pallas import tpu_sc as plsc

N = 10000
NR = 8
R = 2 * NR + 1
E = 320000
D = 128
F = 16

NS = 16
NC = 2
CH = 128
CPT = 157
EPAD = NS * CPT * CH
NCH = EPAD // CH
BK = NR * N
BKPAD = 81920
ZPT = BKPAD // NS
OPT = BK // NS
SR = 1024
OCH = (SR, SR, SR, SR, OPT - 4 * SR)
TN = 1000


def _mm1_body(emb_ref, w_ref, out_ref):
    m = jnp.dot(emb_ref[...], w_ref[...], preferred_element_type=jnp.float32)
    for r in range(R):
        out_ref[r] = m[:, r * F:(r + 1) * F]


def _comb_mm2_body(bk_ref, cnt_ref, self_ref, b1_ref, w2_ref, out_ref):
    acc = self_ref[0] + b1_ref[...]
    for q in range(2 * NR):
        c = cnt_ref[q]
        acc = acc + jnp.where(c > 0, 1.0 / c, 0.0) * bk_ref[q]
    h = jnp.maximum(acc, 0.0)
    m = jnp.dot(h, w2_ref[...], preferred_element_type=jnp.float32)
    for r in range(R):
        out_ref[r] = m[:, r * F:(r + 1) * F]


def _comb2_body(bk_ref, cnt_ref, self_ref, b2_ref, out_ref):
    acc = self_ref[0] + b2_ref[...]
    for q in range(2 * NR):
        c = cnt_ref[q]
        acc = acc + jnp.where(c > 0, 1.0 / c, 0.0) * bk_ref[q]
    out_ref[...] = acc


def _mm1(emb, w1cat):
    return pl.pallas_call(
        _mm1_body,
        grid=(N // TN,),
        in_specs=[
            pl.BlockSpec((TN, D), lambda i: (i, 0)),
            pl.BlockSpec((D, R * F), lambda i: (0, 0)),
        ],
        out_specs=pl.BlockSpec((R, TN, F), lambda i: (0, i, 0)),
        out_shape=jax.ShapeDtypeStruct((R, N, F), jnp.float32),
    )(emb, w1cat)


def _comb_mm2(bk, cnt, t1, b1, w2cat):
    return pl.pallas_call(
        _comb_mm2_body,
        grid=(N // TN,),
        in_specs=[
            pl.BlockSpec((2 * NR, TN, F), lambda i: (0, i, 0)),
            pl.BlockSpec((2 * NR, TN, 1), lambda i: (0, i, 0)),
            pl.BlockSpec((1, TN, F), lambda i: (R - 1, i, 0)),
            pl.BlockSpec((1, F), lambda i: (0, 0)),
            pl.BlockSpec((F, R * F), lambda i: (0, 0)),
        ],
        out_specs=pl.BlockSpec((R, TN, F), lambda i: (0, i, 0)),
        out_shape=jax.ShapeDtypeStruct((R, N, F), jnp.float32),
    )(bk, cnt, t1, b1, w2cat)


def _comb2(bk, cnt, t2, b2):
    return pl.pallas_call(
        _comb2_body,
        grid=(N // TN,),
        in_specs=[
            pl.BlockSpec((2 * NR, TN, F), lambda i: (0, i, 0)),
            pl.BlockSpec((2 * NR, TN, 1), lambda i: (0, i, 0)),
            pl.BlockSpec((1, TN, F), lambda i: (R - 1, i, 0)),
            pl.BlockSpec((1, F), lambda i: (0, 0)),
        ],
        out_specs=pl.BlockSpec((TN, F), lambda i: (i, 0)),
        out_shape=jax.ShapeDtypeStruct((N, F), jnp.float32),
    )(bk, cnt, t2, b2)


def _sc_agg_body(with_counts, *refs):
    if with_counts:
        (table, gidx, bidx, bk_out, cnt_out,
         bucket_sh, cnt_sh, idxg_v, idxb_v, rows_v, ones_v, stage_v,
         stage1_v, sem) = refs
    else:
        (table, gidx, bidx, bk_out,
         bucket_sh, idxg_v, idxb_v, rows_v, stage_v, sem) = refs
    c = lax.axis_index("c")
    t = lax.axis_index("s")

    zrow = jnp.zeros((F,), jnp.float32)

    def zero_body(i, carry):
        stage_v[i] = zrow
        return carry

    lax.fori_loop(0, SR, zero_body, 0)
    for k in range(ZPT // SR):
        pltpu.sync_copy(stage_v, bucket_sh.at[pl.ds(t * ZPT + k * SR, SR)])
    if with_counts:
        def zero1_body(i, carry):
            stage1_v[pl.ds(i * F, F)] = zrow
            return carry

        lax.fori_loop(0, SR // F, zero1_body, 0)
        for k in range(ZPT // SR):
            pltpu.sync_copy(stage1_v, cnt_sh.at[pl.ds(t * ZPT + k * SR, SR)])
        one16 = jnp.ones((16,), jnp.float32)
        for i in range(CH // 16):
            ones_v[pl.ds(i * 16, 16)] = one16
    plsc.subcore_barrier()

    ebase = c * EPAD + t * CPT * CH

    def chunk_body(j, carry):
        pltpu.sync_copy(gidx.at[pl.ds(ebase + j * CH, CH)], idxg_v)
        pltpu.sync_copy(bidx.at[pl.ds(ebase + j * CH, CH)], idxb_v)
        pltpu.async_copy(table.at[idxg_v], rows_v, sem).wait()
        pltpu.sync_copy(rows_v, bucket_sh.at[idxb_v], add=True)
        if with_counts:
            pltpu.sync_copy(ones_v, cnt_sh.at[idxb_v], add=True)
        return carry

    lax.fori_loop(0, CPT, chunk_body, 0)
    plsc.subcore_barrier()

    off = 0
    for sz in OCH:
        pltpu.sync_copy(bucket_sh.at[pl.ds(t * OPT + off, sz)],
                        stage_v.at[pl.ds(0, sz)])
        pltpu.sync_copy(stage_v.at[pl.ds(0, sz)],
                        bk_out.at[pl.ds(c * BK + t * OPT + off, sz)])
        if with_counts:
            pltpu.sync_copy(cnt_sh.at[pl.ds(t * OPT + off, sz)],
                            stage1_v.at[pl.ds(0, sz)])
            pltpu.sync_copy(stage1_v.at[pl.ds(0, sz)],
                            cnt_out.at[pl.ds(c * BK + t * OPT + off, sz)])
        off += sz


def _sc_mesh():
    return plsc.VectorSubcoreMesh(
        core_axis_name="c", subcore_axis_name="s", num_cores=NC, num_subcores=NS)


def _sc_agg1(table, gidx, bidx):
    return pl.kernel(
        functools.partial(_sc_agg_body, True),
        out_type=(
            jax.ShapeDtypeStruct((NC * BK, F), jnp.float32),
            jax.ShapeDtypeStruct((NC * BK,), jnp.float32),
        ),
        mesh=_sc_mesh(),
        scratch_types=[
            pltpu.VMEM_SHARED((BKPAD, F), jnp.float32),
            pltpu.VMEM_SHARED((BKPAD,), jnp.float32),
            pltpu.VMEM((CH,), jnp.int32),
            pltpu.VMEM((CH,), jnp.int32),
            pltpu.VMEM((CH, F), jnp.float32),
            pltpu.VMEM((CH,), jnp.float32),
            pltpu.VMEM((SR, F), jnp.float32),
            pltpu.VMEM((SR,), jnp.float32),
            pltpu.SemaphoreType.DMA,
        ],
        compiler_params=pltpu.CompilerParams(use_tc_tiling_on_sc=False),
    )(table, gidx, bidx)


def _sc_agg2(table, gidx, bidx):
    return pl.kernel(
        functools.partial(_sc_agg_body, False),
        out_type=jax.ShapeDtypeStruct((NC * BK, F), jnp.float32),
        mesh=_sc_mesh(),
        scratch_types=[
            pltpu.VMEM_SHARED((BKPAD, F), jnp.float32),
            pltpu.VMEM((CH,), jnp.int32),
            pltpu.VMEM((CH,), jnp.int32),
            pltpu.VMEM((CH, F), jnp.float32),
            pltpu.VMEM((SR, F), jnp.float32),
            pltpu.SemaphoreType.DMA,
        ],
        compiler_params=pltpu.CompilerParams(use_tc_tiling_on_sc=False),
    )(table, gidx, bidx)


def kernel(edge_nodes, edge_rels, node_embeddings, weights1, weights2, bias1, bias2):
    s, o = edge_nodes[0], edge_nodes[1]
    p = edge_rels

    g0 = p * N + o
    b0 = p * N + s
    g1 = (p + NR) * N + s
    b1 = p * N + o
    padlen = EPAD - E
    padg = jnp.arange(padlen, dtype=jnp.int32) % CH
    padb = BK + jnp.arange(padlen, dtype=jnp.int32) % (BKPAD - BK)
    gidx = jnp.concatenate([g0, padg, g1, padg])
    bidx = jnp.concatenate([b0, padb, b1, padb])

    w1cat = jnp.transpose(weights1, (1, 0, 2)).reshape(D, R * F)
    w2cat = jnp.transpose(weights2, (1, 0, 2)).reshape(F, R * F)
    b1_2d = bias1.reshape(1, F)
    b2_2d = bias2.reshape(1, F)

    t1 = _mm1(node_embeddings, w1cat)
    bk1, cnt = _sc_agg1(t1.reshape(R * N, F), gidx, bidx)
    cnt3d = cnt.reshape(2 * NR, N, 1)
    t2 = _comb_mm2(bk1.reshape(2 * NR, N, F), cnt3d, t1, b1_2d, w2cat)
    bk2 = _sc_agg2(t2.reshape(R * N, F), gidx, bidx)
    out = _comb2(bk2.reshape(2 * NR, N, F), cnt3d, t2, b2_2d)
    return out


# speedup vs baseline: 36.1452x; 1.5742x over previous
import functools

import jax
import jax.numpy as jnp
from jax import lax
from jax.experimental import pallas as pl
from jax.experimental.pallas import tpu as pltpu
from jax.experimental.pallas import tpu_sc as plsc

N = 10000
NR = 8
R = 2 * NR + 1
E = 320000
D = 128
F = 16

NS = 16
NC = 2
CH = 128
CPT = 160
NB = 40
NBATCH = CPT // NB
EPAD = NS * CPT * CH
NCH = EPAD // CH
BK = NR * N
BKPAD = 81920
ZPT = BKPAD // NS
OPT = BK // NS
SR = 1024
OCH = (SR, SR, SR, SR, OPT - 4 * SR)
TN = 1000


def _mm1_body(emb_ref, w_ref, out_ref):
    m = jnp.dot(emb_ref[...], w_ref[...], preferred_element_type=jnp.float32)
    for r in range(R):
        out_ref[r] = m[:, r * F:(r + 1) * F]


def _comb_mm2_body(bk_ref, cnt_ref, self_ref, b1_ref, w2_ref, out_ref):
    acc = self_ref[0] + b1_ref[...]
    for q in range(2 * NR):
        c = cnt_ref[q]
        acc = acc + jnp.where(c > 0, 1.0 / c, 0.0) * bk_ref[q]
    h = jnp.maximum(acc, 0.0)
    m = jnp.dot(h, w2_ref[...], preferred_element_type=jnp.float32)
    for r in range(R):
        out_ref[r] = m[:, r * F:(r + 1) * F]


def _comb2_body(bk_ref, cnt_ref, self_ref, b2_ref, out_ref):
    acc = self_ref[0] + b2_ref[...]
    for q in range(2 * NR):
        c = cnt_ref[q]
        acc = acc + jnp.where(c > 0, 1.0 / c, 0.0) * bk_ref[q]
    out_ref[...] = acc


def _mm1(emb, w1cat):
    return pl.pallas_call(
        _mm1_body,
        grid=(N // TN,),
        in_specs=[
            pl.BlockSpec((TN, D), lambda i: (i, 0)),
            pl.BlockSpec((D, R * F), lambda i: (0, 0)),
        ],
        out_specs=pl.BlockSpec((R, TN, F), lambda i: (0, i, 0)),
        out_shape=jax.ShapeDtypeStruct((R, N, F), jnp.float32),
    )(emb, w1cat)


def _comb_mm2(bk, cnt, t1, b1, w2cat):
    return pl.pallas_call(
        _comb_mm2_body,
        grid=(N // TN,),
        in_specs=[
            pl.BlockSpec((2 * NR, TN, F), lambda i: (0, i, 0)),
            pl.BlockSpec((2 * NR, TN, 1), lambda i: (0, i, 0)),
            pl.BlockSpec((1, TN, F), lambda i: (R - 1, i, 0)),
            pl.BlockSpec((1, F), lambda i: (0, 0)),
            pl.BlockSpec((F, R * F), lambda i: (0, 0)),
        ],
        out_specs=pl.BlockSpec((R, TN, F), lambda i: (0, i, 0)),
        out_shape=jax.ShapeDtypeStruct((R, N, F), jnp.float32),
    )(bk, cnt, t1, b1, w2cat)


def _comb2(bk, cnt, t2, b2):
    return pl.pallas_call(
        _comb2_body,
        grid=(N // TN,),
        in_specs=[
            pl.BlockSpec((2 * NR, TN, F), lambda i: (0, i, 0)),
            pl.BlockSpec((2 * NR, TN, 1), lambda i: (0, i, 0)),
            pl.BlockSpec((1, TN, F), lambda i: (R - 1, i, 0)),
            pl.BlockSpec((1, F), lambda i: (0, 0)),
        ],
        out_specs=pl.BlockSpec((TN, F), lambda i: (i, 0)),
        out_shape=jax.ShapeDtypeStruct((N, F), jnp.float32),
    )(bk, cnt, t2, b2)


def _sc_agg_body(with_counts, *refs):
    if with_counts:
        (table, gidx, bidx, bk_out, cnt_out,
         bucket_sh, cnt_sh, idxg_v, idxb_v, rows0_v, rows1_v, ones_v, stage_v,
         stage1_v, sem0, sem1) = refs
    else:
        (table, gidx, bidx, bk_out,
         bucket_sh, idxg_v, idxb_v, rows0_v, rows1_v, stage_v, sem0, sem1) = refs
    c = lax.axis_index("c")
    t = lax.axis_index("s")

    zrow = jnp.zeros((F,), jnp.float32)

    def zero_body(i, carry):
        stage_v[i] = zrow
        return carry

    lax.fori_loop(0, SR, zero_body, 0)
    for k in range(ZPT // SR):
        pltpu.sync_copy(stage_v, bucket_sh.at[pl.ds(t * ZPT + k * SR, SR)])
    if with_counts:
        def zero1_body(i, carry):
            stage1_v[pl.ds(i * F, F)] = zrow
            return carry

        lax.fori_loop(0, SR // F, zero1_body, 0)
        for k in range(ZPT // SR):
            pltpu.sync_copy(stage1_v, cnt_sh.at[pl.ds(t * ZPT + k * SR, SR)])
        one16 = jnp.ones((16,), jnp.float32)
        for i in range(CH // 16):
            ones_v[pl.ds(i * 16, 16)] = one16
    plsc.subcore_barrier()

    rowbase = c * (NCH) + t * CPT

    def _wait(rows_v, sem):
        pltpu.make_async_copy(table.at[pl.ds(0, CH)], rows_v, sem).wait()

    for k in range(NBATCH):
        pltpu.sync_copy(gidx.at[pl.ds(rowbase + k * NB, NB)], idxg_v)
        pltpu.sync_copy(bidx.at[pl.ds(rowbase + k * NB, NB)], idxb_v)
        pltpu.async_copy(table.at[idxg_v.at[0]], rows0_v, sem0)
        pltpu.async_copy(table.at[idxg_v.at[1]], rows1_v, sem1)

        def pair_body(i, carry):
            c0 = 2 * i
            c1 = c0 + 1
            _wait(rows0_v, sem0)
            pltpu.sync_copy(rows0_v, bucket_sh.at[idxb_v.at[c0]], add=True)
            if with_counts:
                pltpu.sync_copy(ones_v, cnt_sh.at[idxb_v.at[c0]], add=True)

            @pl.when(c0 + 2 < NB)
            def _():
                pltpu.async_copy(table.at[idxg_v.at[c0 + 2]], rows0_v, sem0)

            _wait(rows1_v, sem1)
            pltpu.sync_copy(rows1_v, bucket_sh.at[idxb_v.at[c1]], add=True)
            if with_counts:
                pltpu.sync_copy(ones_v, cnt_sh.at[idxb_v.at[c1]], add=True)

            @pl.when(c1 + 2 < NB)
            def _():
                pltpu.async_copy(table.at[idxg_v.at[c1 + 2]], rows1_v, sem1)

            return carry

        lax.fori_loop(0, NB // 2, pair_body, 0)

    plsc.subcore_barrier()

    off = 0
    for sz in OCH:
        pltpu.sync_copy(bucket_sh.at[pl.ds(t * OPT + off, sz)],
                        stage_v.at[pl.ds(0, sz)])
        pltpu.sync_copy(stage_v.at[pl.ds(0, sz)],
                        bk_out.at[pl.ds(c * BK + t * OPT + off, sz)])
        if with_counts:
            pltpu.sync_copy(cnt_sh.at[pl.ds(t * OPT + off, sz)],
                            stage1_v.at[pl.ds(0, sz)])
            pltpu.sync_copy(stage1_v.at[pl.ds(0, sz)],
                            cnt_out.at[pl.ds(c * BK + t * OPT + off, sz)])
        off += sz


def _sc_mesh():
    return plsc.VectorSubcoreMesh(
        core_axis_name="c", subcore_axis_name="s", num_cores=NC, num_subcores=NS)


def _sc_agg1(table, gidx, bidx):
    return pl.kernel(
        functools.partial(_sc_agg_body, True),
        out_type=(
            jax.ShapeDtypeStruct((NC * BK, F), jnp.float32),
            jax.ShapeDtypeStruct((NC * BK,), jnp.float32),
        ),
        mesh=_sc_mesh(),
        scratch_types=[
            pltpu.VMEM_SHARED((BKPAD, F), jnp.float32),
            pltpu.VMEM_SHARED((BKPAD,), jnp.float32),
            pltpu.VMEM((NB, CH), jnp.int32),
            pltpu.VMEM((NB, CH), jnp.int32),
            pltpu.VMEM((CH, F), jnp.float32),
            pltpu.VMEM((CH, F), jnp.float32),
            pltpu.VMEM((CH,), jnp.float32),
            pltpu.VMEM((SR, F), jnp.float32),
            pltpu.VMEM((SR,), jnp.float32),
            pltpu.SemaphoreType.DMA,
            pltpu.SemaphoreType.DMA,
        ],
        compiler_params=pltpu.CompilerParams(use_tc_tiling_on_sc=False),
    )(table, gidx, bidx)


def _sc_agg2(table, gidx, bidx):
    return pl.kernel(
        functools.partial(_sc_agg_body, False),
        out_type=jax.ShapeDtypeStruct((NC * BK, F), jnp.float32),
        mesh=_sc_mesh(),
        scratch_types=[
            pltpu.VMEM_SHARED((BKPAD, F), jnp.float32),
            pltpu.VMEM((NB, CH), jnp.int32),
            pltpu.VMEM((NB, CH), jnp.int32),
            pltpu.VMEM((CH, F), jnp.float32),
            pltpu.VMEM((CH, F), jnp.float32),
            pltpu.VMEM((SR, F), jnp.float32),
            pltpu.SemaphoreType.DMA,
            pltpu.SemaphoreType.DMA,
        ],
        compiler_params=pltpu.CompilerParams(use_tc_tiling_on_sc=False),
    )(table, gidx, bidx)


def kernel(edge_nodes, edge_rels, node_embeddings, weights1, weights2, bias1, bias2):
    s, o = edge_nodes[0], edge_nodes[1]
    p = edge_rels

    g0 = p * N + o
    b0 = p * N + s
    g1 = (p + NR) * N + s
    b1 = p * N + o
    padlen = EPAD - E
    padg = jnp.arange(padlen, dtype=jnp.int32) % CH
    padb = BK + jnp.arange(padlen, dtype=jnp.int32) % (BKPAD - BK)
    gidx = jnp.concatenate([g0, padg, g1, padg]).reshape(NC * NCH, CH)
    bidx = jnp.concatenate([b0, padb, b1, padb]).reshape(NC * NCH, CH)

    w1cat = jnp.transpose(weights1, (1, 0, 2)).reshape(D, R * F)
    w2cat = jnp.transpose(weights2, (1, 0, 2)).reshape(F, R * F)
    b1_2d = bias1.reshape(1, F)
    b2_2d = bias2.reshape(1, F)

    t1 = _mm1(node_embeddings, w1cat)
    bk1, cnt = _sc_agg1(t1.reshape(R * N, F), gidx, bidx)
    cnt3d = cnt.reshape(2 * NR, N, 1)
    t2 = _comb_mm2(bk1.reshape(2 * NR, N, F), cnt3d, t1, b1_2d, w2cat)
    bk2 = _sc_agg2(t2.reshape(R * N, F), gidx, bidx)
    out = _comb2(bk2.reshape(2 * NR, N, F), cnt3d, t2, b2_2d)
    return out
